# baseline (speedup 1.0000x reference)
import functools

import jax
import jax.numpy as jnp
from jax import lax
from jax.experimental import pallas as pl
from jax.experimental.pallas import tpu as pltpu
from jax.experimental.pallas import tpu_sc as plsc

_D = 64
_B = 16384
_NC = 2
_NS = 16
_NW = _NC * _NS
_BPW = _B // _NW
_CH = 128
_NCH = _BPW // _CH
_L = 16
_G = _D // _L
_NQ = 5


def _sc_partials(idx_hbm, table_hbm, out_hbm, idxo_v, idx2_v, offs_v,
                 x_v, x1_v, x2_v, acc_v, sem):
    wid = lax.axis_index("s") * _NC + lax.axis_index("c")
    for t in range(3):
        pltpu.sync_copy(idx_hbm.at[t, wid], idxo_v.at[pl.ds(t * _NCH, _NCH)])
    for r in range(3 * _NCH):
        for k in range(_CH // _L):
            v = idxo_v[r, pl.ds(k * _L, _L)]
            idx2_v[r, pl.ds(k * _L, _L)] = lax.shift_right_logical(v, 1)
            offs_v[r, pl.ds(k * _L, _L)] = lax.shift_left(
                lax.bitwise_and(v, 1), 6)

    rows = (x_v, x1_v, x2_v)
    zeros = jnp.zeros((_L,), jnp.float32)
    accs = (zeros,) * (_NQ * _G)

    for j in range(_NCH):
        copies = [
            pltpu.async_copy(table_hbm.at[idx2_v.at[t * _NCH + j]],
                             rows[t], sem)
            for t in range(3)
        ]
        for c in copies:
            c.wait()

        def body(grp, a, j=j):
            base = grp * _L
            ov0 = offs_v[0 * _NCH + j, pl.ds(base, _L)]
            ov1 = offs_v[1 * _NCH + j, pl.ds(base, _L)]
            ov2 = offs_v[2 * _NCH + j, pl.ds(base, _L)]
            a = list(a)
            for i in range(_L):
                b = base + i
                o0, o1, o2 = ov0[i], ov1[i], ov2[i]
                for g in range(_G):
                    x = x_v[b, pl.ds(o0 + g * _L, _L)]
                    x1 = x1_v[b, pl.ds(o1 + g * _L, _L)]
                    x2 = x2_v[b, pl.ds(o2 + g * _L, _L)]
                    q = g * _NQ
                    a[q] += x * x1
                    a[q + 1] += x * x2
                    a[q + 2] += x * x
                    a[q + 3] += x1 * x1
                    a[q + 4] += x2 * x2
            return tuple(a)

        accs = lax.fori_loop(0, _CH // _L, body, accs)

    for g in range(_G):
        for q in range(_NQ):
            acc_v[q, pl.ds(g * _L, _L)] = accs[g * _NQ + q]
    pltpu.sync_copy(acc_v, out_hbm.at[wid])


def _tc_combine(p_ref, o_ref):
    s = jnp.sum(p_ref[...], axis=0)
    num1 = s[0:1, :]
    num2 = s[1:2, :]
    nxx = jnp.sqrt(s[2:3, :])
    n11 = jnp.sqrt(s[3:4, :])
    n22 = jnp.sqrt(s[4:5, :])
    one = num1 / jnp.maximum(nxx * n11, 1e-6)
    two = num2 / jnp.maximum(nxx * n22, 1e-6)
    o_ref[...] = two - one


@jax.jit
def kernel(inputs_, embeddings):
    idx = inputs_.reshape(3, _NW, _NCH, _CH)
    table = embeddings.reshape(500000, 128)

    mesh = plsc.VectorSubcoreMesh(core_axis_name="c", subcore_axis_name="s")
    partials = pl.kernel(
        _sc_partials,
        mesh=mesh,
        out_type=jax.ShapeDtypeStruct((_NW, _NQ, _D), jnp.float32),
        scratch_types=[
            pltpu.VMEM((3 * _NCH, _CH), jnp.int32),
            pltpu.VMEM((3 * _NCH, _CH), jnp.int32),
            pltpu.VMEM((3 * _NCH, _CH), jnp.int32),
            pltpu.VMEM((_CH, 128), jnp.float32),
            pltpu.VMEM((_CH, 128), jnp.float32),
            pltpu.VMEM((_CH, 128), jnp.float32),
            pltpu.VMEM((_NQ, _D), jnp.float32),
            pltpu.SemaphoreType.DMA,
        ],
    )(idx, table)

    out = pl.pallas_call(
        _tc_combine,
        out_shape=jax.ShapeDtypeStruct((1, _D), jnp.float32),
    )(partials)
    return out.reshape(_D)


# speedup vs baseline: 1.4498x; 1.4498x over previous
import functools

import jax
import jax.numpy as jnp
from jax import lax
from jax.experimental import pallas as pl
from jax.experimental.pallas import tpu as pltpu
from jax.experimental.pallas import tpu_sc as plsc

_D = 64
_B = 16384
_NC = 2
_NS = 16
_NW = _NC * _NS
_BPW = _B // _NW
_L = 16
_G = _D // _L
_NQ = 5
_CR = 16
_NCHK = _BPW // _CR


def _row_vec(idx_v, t, c):
    j = lax.shift_right_logical(c, 3)
    sub = lax.bitwise_and(c, 7) * _L
    return idx_v[t, j, pl.ds(sub, _L)]


def _sc_partials(idx_hbm, table_hbm, out_hbm, idx_v, blk_a, blk_b,
                 acc_v, sem_a, sem_b):
    wid = lax.axis_index("s") * _NC + lax.axis_index("c")
    for t in range(3):
        pltpu.sync_copy(idx_hbm.at[t, wid], idx_v.at[t])

    def fire(c, bufs, sem):
        for t in range(3):
            rv = _row_vec(idx_v, t, c)
            for i in range(_L):
                r8 = pl.multiple_of(
                    lax.shift_left(lax.shift_right_logical(rv[i], 3), 3),
                    8)
                pltpu.async_copy(table_hbm.at[pl.ds(r8, 8)],
                                 bufs[t].at[i], sem)

    def drain(bufs, sem):
        for t in range(3):
            for i in range(_CR):
                pltpu.make_async_copy(table_hbm.at[pl.ds(0, 8)],
                                      bufs[t].at[i], sem).wait()

    def compute(c, bufs, a):
        a = list(a)
        sv0 = lax.bitwise_and(_row_vec(idx_v, 0, c), 7)
        sv1 = lax.bitwise_and(_row_vec(idx_v, 1, c), 7)
        sv2 = lax.bitwise_and(_row_vec(idx_v, 2, c), 7)
        for i in range(_L):
            s0, s1, s2 = sv0[i], sv1[i], sv2[i]
            for g in range(_G):
                x = bufs[0][i, s0, pl.ds(g * _L, _L)]
                x1 = bufs[1][i, s1, pl.ds(g * _L, _L)]
                x2 = bufs[2][i, s2, pl.ds(g * _L, _L)]
                q = g * _NQ
                a[q] += x * x1
                a[q + 1] += x * x2
                a[q + 2] += x * x
                a[q + 3] += x1 * x1
                a[q + 4] += x2 * x2
        return tuple(a)

    zeros = jnp.zeros((_L,), jnp.float32)
    fire(0, blk_a, sem_a)

    def body(u, a):
        c = u * 2
        fire(c + 1, blk_b, sem_b)
        drain(blk_a, sem_a)
        a = compute(c, blk_a, a)
        fire(lax.min(c + 2, _NCHK - 1), blk_a, sem_a)
        drain(blk_b, sem_b)
        a = compute(c + 1, blk_b, a)
        return a

    accs = lax.fori_loop(0, _NCHK // 2, body, (zeros,) * (_NQ * _G))
    drain(blk_a, sem_a)

    for g in range(_G):
        for q in range(_NQ):
            acc_v[q, pl.ds(g * _L, _L)] = accs[g * _NQ + q]
    pltpu.sync_copy(acc_v, out_hbm.at[wid])


def _tc_combine(p_ref, o_ref):
    s = jnp.sum(p_ref[...], axis=0)
    num1 = s[0:1, :]
    num2 = s[1:2, :]
    nxx = jnp.sqrt(s[2:3, :])
    n11 = jnp.sqrt(s[3:4, :])
    n22 = jnp.sqrt(s[4:5, :])
    one = num1 / jnp.maximum(nxx * n11, 1e-6)
    two = num2 / jnp.maximum(nxx * n22, 1e-6)
    o_ref[...] = two - one


@jax.jit
def kernel(inputs_, embeddings):
    idx = inputs_.reshape(3, _NW, _BPW // 128, 128)

    mesh = plsc.VectorSubcoreMesh(core_axis_name="c", subcore_axis_name="s")
    blk = pltpu.VMEM((_CR, 8, _D), jnp.float32)
    partials = pl.kernel(
        _sc_partials,
        mesh=mesh,
        out_type=jax.ShapeDtypeStruct((_NW, _NQ, _D), jnp.float32),
        scratch_types=[
            pltpu.VMEM((3, _BPW // 128, 128), jnp.int32),
            (blk, blk, blk),
            (blk, blk, blk),
            pltpu.VMEM((_NQ, _D), jnp.float32),
            pltpu.SemaphoreType.DMA,
            pltpu.SemaphoreType.DMA,
        ],
    )(idx, embeddings)

    out = pl.pallas_call(
        _tc_combine,
        out_shape=jax.ShapeDtypeStruct((1, _D), jnp.float32),
    )(partials)
    return out.reshape(_D)


# speedup vs baseline: 2.0330x; 1.4023x over previous
import jax
import jax.numpy as jnp
from jax import lax
from jax.experimental import pallas as pl
from jax.experimental.pallas import tpu as pltpu
from jax.experimental.pallas import tpu_sc as plsc

_D = 64
_B = 16384
_NC = 2
_NS = 16
_NW = _NC * _NS
_BPW = _B // _NW
_L = 16
_G = _D // _L
_NQ = 5
_CR = 8
_NCHK = _BPW // _CR


def _row_vec(idx_v, t, c):
    j = lax.shift_right_logical(c, 4)
    sub = lax.bitwise_and(lax.shift_right_logical(c, 1), 7) * _L
    return idx_v[t, j, pl.ds(sub, _L)]


def _sc_partials(idx_hbm, table_hbm, out_hbm, idx_v, blk_a, blk_b,
                 blk_c, blk_d, acc_v, sem_a, sem_b, sem_c, sem_d):
    wid = lax.axis_index("s") * _NC + lax.axis_index("c")
    for t in range(3):
        pltpu.sync_copy(idx_hbm.at[t, wid], idx_v.at[t])

    def fire(c, bufs, sem, lo):
        for t in range(3):
            rv = _row_vec(idx_v, t, c)
            for i in range(_CR):
                q = lax.shift_right_logical(rv[lo + i], 3)
                pltpu.async_copy(table_hbm.at[q], bufs[t].at[i], sem)

    def drain(bufs, sem):
        for t in range(3):
            for i in range(_CR):
                pltpu.make_async_copy(table_hbm.at[0],
                                      bufs[t].at[i], sem).wait()

    def compute(c, bufs, a, lo):
        a = list(a)
        sv0 = lax.bitwise_and(_row_vec(idx_v, 0, c), 7)
        sv1 = lax.bitwise_and(_row_vec(idx_v, 1, c), 7)
        sv2 = lax.bitwise_and(_row_vec(idx_v, 2, c), 7)
        for i in range(_CR):
            s0, s1, s2 = sv0[lo + i], sv1[lo + i], sv2[lo + i]
            for g in range(_G):
                x = bufs[0][i, s0, pl.ds(g * _L, _L)]
                x1 = bufs[1][i, s1, pl.ds(g * _L, _L)]
                x2 = bufs[2][i, s2, pl.ds(g * _L, _L)]
                q = g * _NQ
                a[q] += x * x1
                a[q + 1] += x * x2
                a[q + 2] += x * x
                a[q + 3] += x1 * x1
                a[q + 4] += x2 * x2
        return tuple(a)

    zeros = jnp.zeros((_L,), jnp.float32)
    sets = ((blk_a, sem_a), (blk_b, sem_b), (blk_c, sem_c), (blk_d, sem_d))
    for k in range(3):
        fire(k, sets[k][0], sets[k][1], (k & 1) * _CR)

    def body(u, a):
        c0 = u * 4
        for k in range(4):
            c = c0 + k
            bufs, sem = sets[k]
            drain(bufs, sem)
            a = compute(c, bufs, a, (k & 1) * _CR)
            nbufs, nsem = sets[(k + 3) % 4]
            fire(lax.min(c + 3, _NCHK - 1), nbufs, nsem,
                 ((k + 3) & 1) * _CR)
        return a

    accs = lax.fori_loop(0, _NCHK // 4, body, (zeros,) * (_NQ * _G))
    for k in range(3):
        drain(sets[k][0], sets[k][1])

    for g in range(_G):
        for q in range(_NQ):
            acc_v[q, pl.ds(g * _L, _L)] = accs[g * _NQ + q]
    pltpu.sync_copy(acc_v, out_hbm.at[wid])


def _tc_combine(p_ref, o_ref):
    s = jnp.sum(p_ref[...], axis=0)
    num1 = s[0:1, :]
    num2 = s[1:2, :]
    nxx = jnp.sqrt(s[2:3, :])
    n11 = jnp.sqrt(s[3:4, :])
    n22 = jnp.sqrt(s[4:5, :])
    one = num1 / jnp.maximum(nxx * n11, 1e-6)
    two = num2 / jnp.maximum(nxx * n22, 1e-6)
    o_ref[...] = two - one


@jax.jit
def kernel(inputs_, embeddings):
    idx = inputs_.reshape(3, _NW, _BPW // 128, 128)

    mesh = plsc.VectorSubcoreMesh(core_axis_name="c", subcore_axis_name="s")
    blk = pltpu.VMEM((_CR, 8, _D), jnp.float32)
    partials = pl.kernel(
        _sc_partials,
        mesh=mesh,
        out_type=jax.ShapeDtypeStruct((_NW, _NQ, _D), jnp.float32),
        scratch_types=[
            pltpu.VMEM((3, _BPW // 128, 128), jnp.int32),
            (blk, blk, blk),
            (blk, blk, blk),
            (blk, blk, blk),
            (blk, blk, blk),
            pltpu.VMEM((_NQ, _D), jnp.float32),
            pltpu.SemaphoreType.DMA,
            pltpu.SemaphoreType.DMA,
            pltpu.SemaphoreType.DMA,
            pltpu.SemaphoreType.DMA,
        ],
    )(idx, embeddings.reshape(125000, 8, _D))

    out = pl.pallas_call(
        _tc_combine,
        out_shape=jax.ShapeDtypeStruct((1, _D), jnp.float32),
    )(partials)
    return out.reshape(_D)
